# baseline (speedup 1.0000x reference)
import jax
import jax.numpy as jnp
from jax import lax
from jax.experimental import pallas as pl
from jax.experimental.pallas import tpu as pltpu
from jax.experimental.pallas import tpu_sc as plsc

_B, _S, _P = 16, 4096, 8192
_NC, _NS, _L = 2, 16, 16
_NW = _NC * _NS
_PPW = _B * _P // _NW
_W = 2 * _PPW
_UNROLL = 8
_ITERS = _W // (_L * _UNROLL)
_SCALE = 0.5 / (_B * _S)


def _dtw_body(preds_hbm, targets_hbm, paths_hbm, out_hbm,
              cat_v, path_v, acc_v, sem_p, sem_t, sem_i):
    wid = lax.axis_index("s") * _NC + lax.axis_index("c")
    b = wid // 2

    cp_p = pltpu.make_async_copy(preds_hbm.at[b], cat_v.at[pl.ds(0, 2 * _S)],
                                 sem_p)
    cp_t = pltpu.make_async_copy(targets_hbm.at[b],
                                 cat_v.at[pl.ds(2 * _S, 2 * _S)], sem_t)
    cp_i = pltpu.make_async_copy(paths_hbm.at[pl.ds(wid * _W, _W)], path_v,
                                 sem_i)
    cp_p.start()
    cp_t.start()
    cp_i.start()
    cp_p.wait()
    cp_t.wait()
    cp_i.wait()

    lanes = lax.iota(jnp.int32, _L)
    par_off = (lanes & 1) * (2 * _S)
    swap = lanes ^ 1

    def step(k, acc):
        kbase = k * (_L * _UNROLL)
        for u in range(_UNROLL):
            off = kbase + u * _L
            v = path_v[pl.ds(off, _L)]
            ax = v * 2 + par_off
            gx = plsc.load_gather(cat_v, [ax])
            gy = plsc.load_gather(cat_v, [ax + 1])
            dx = gx - jnp.take_along_axis(gx, swap, axis=0)
            dy = gy - jnp.take_along_axis(gy, swap, axis=0)
            acc = acc + (jnp.abs(dx) + jnp.abs(dy))
        return acc

    acc = lax.fori_loop(0, _ITERS, step, jnp.zeros((_L,), jnp.float32))
    acc_v[...] = acc * _SCALE
    pltpu.sync_copy(acc_v, out_hbm.at[wid])


def kernel(preds, targets, paths):
    preds2 = preds.reshape(_B, _S * 2)
    targets2 = targets.reshape(_B, _S * 2)
    paths1 = paths.reshape(_B * _P * 2)
    partials = pl.kernel(
        _dtw_body,
        out_type=jax.ShapeDtypeStruct((_NW, _L), jnp.float32),
        mesh=plsc.VectorSubcoreMesh(core_axis_name="c", subcore_axis_name="s"),
        compiler_params=pltpu.CompilerParams(needs_layout_passes=False),
        scratch_types=[
            pltpu.VMEM((4 * _S,), jnp.float32),
            pltpu.VMEM((_W,), jnp.int32),
            pltpu.VMEM((_L,), jnp.float32),
            pltpu.SemaphoreType.DMA,
            pltpu.SemaphoreType.DMA,
            pltpu.SemaphoreType.DMA,
        ],
    )(preds2, targets2, paths1)
    return jnp.sum(partials)
